# Initial kernel scaffold; baseline (speedup 1.0000x reference)
import jax
import jax.numpy as jnp
from jax.experimental import pallas as pl


def kernel(node_feat, x_t, t, edge_index, edge_attr, batch_ids, movable_mask, params):
    raise NotImplementedError("write your pallas kernel here")


# speedup vs baseline: 2.7844x; 2.7844x over previous
import functools

import jax
import jax.numpy as jnp
from jax import lax
from jax.experimental import pallas as pl
from jax.experimental.pallas import tpu as pltpu
from jax.experimental.pallas import tpu_sc as plsc

N = 10000
E = 320000
G = 16
NODE_DIM = 128
EDGE_DIM = 4
HID = 64
TIME_DIM = 64
HALF = TIME_DIM // 2

NW = 32
EPW = E // NW
EK = 80
NCHUNK = EPW // EK
NP = 10240
RPT = NP // 16
LANES = 16


def _silu(x):
    return x / (1.0 + jnp.exp(-x))


def _pre_body(t_ref, bid_ref, nf_ref, xt_ref,
              tw1a, tw1b, tb1, tw2, tb2,
              e1a, e1b, e1c, eb1, e2, eb2,
              ws0, wd0,
              h_ref, a_ref, b_ref):
    bid = bid_ref[...]
    gids = lax.broadcasted_iota(jnp.int32, (N, G), 1)
    onehot = (bid == gids).astype(jnp.float32)
    t_node = jnp.dot(onehot, t_ref[...],
                     preferred_element_type=jnp.float32)
    k = lax.broadcasted_iota(jnp.int32, (1, HALF), 1).astype(jnp.float32)
    freqs = jnp.exp(k * (-jnp.log(10000.0) / HALF))
    ang = t_node * freqs
    s = jnp.sin(ang)
    c = jnp.cos(ang)
    th = _silu(jnp.dot(s, tw1a[...], preferred_element_type=jnp.float32)
               + jnp.dot(c, tw1b[...], preferred_element_type=jnp.float32)
               + tb1[...])
    te = jnp.dot(th, tw2[...], preferred_element_type=jnp.float32) + tb2[...]
    h1 = _silu(jnp.dot(nf_ref[...], e1a[...], preferred_element_type=jnp.float32)
               + jnp.dot(xt_ref[...], e1b[...], preferred_element_type=jnp.float32)
               + jnp.dot(te, e1c[...], preferred_element_type=jnp.float32)
               + eb1[...])
    h = jnp.dot(h1, e2[...], preferred_element_type=jnp.float32) + eb2[...]
    h_ref[...] = h
    a_ref[...] = jnp.dot(h, ws0[...], preferred_element_type=jnp.float32)
    b_ref[...] = jnp.dot(h, wd0[...], preferred_element_type=jnp.float32)


_BE = 16000

def _c_body(ea_ref, we0, mb0, we1, mb1, c0_ref, c1_ref):
    ea = ea_ref[...]
    c0_ref[...] = jnp.dot(ea, we0[...], preferred_element_type=jnp.float32) + mb0[...]
    c1_ref[...] = jnp.dot(ea, we1[...], preferred_element_type=jnp.float32) + mb1[...]


def _upd_body(h_ref, p0_ref, p1_ref, uh, ua, ub, ws, wd,
              h_ref_out, a_ref, b_ref):
    agg = p0_ref[...] + p1_ref[...]
    hn = _silu(jnp.dot(h_ref[...], uh[...], preferred_element_type=jnp.float32)
               + jnp.dot(agg, ua[...], preferred_element_type=jnp.float32)
               + ub[...])
    h_ref_out[...] = hn
    a_ref[...] = jnp.dot(hn, ws[...], preferred_element_type=jnp.float32)
    b_ref[...] = jnp.dot(hn, wd[...], preferred_element_type=jnp.float32)


def _head_body(h_ref, p0_ref, p1_ref, uh, ua, ub,
               ln_g, ln_b, hw1, hb1, hw2, hb2, mask_ref,
               out_ref):
    agg = p0_ref[...] + p1_ref[...]
    h = _silu(jnp.dot(h_ref[...], uh[...], preferred_element_type=jnp.float32)
              + jnp.dot(agg, ua[...], preferred_element_type=jnp.float32)
              + ub[...])
    mu = jnp.mean(h, axis=1, keepdims=True)
    d = h - mu
    var = jnp.mean(d * d, axis=1, keepdims=True)
    hn = d * lax.rsqrt(var + 1e-5) * ln_g[...] + ln_b[...]
    f1 = _silu(jnp.dot(hn, hw1[...], preferred_element_type=jnp.float32) + hb1[...])
    f = jnp.dot(f1, hw2[...], preferred_element_type=jnp.float32) + hb2[...]
    out_ref[...] = f * mask_ref[...]


def _sc_edge_body(a_hbm, b_hbm, c_hbm, src_hbm, dst_hbm, p_hbm,
                  s_idx, d_idx, a_buf, b_buf, c_buf, tmp, agg, sem_a, sem_b):
    cid = lax.axis_index("c")
    sid = lax.axis_index("s")
    nbase = sid * RPT

    zrow = jnp.zeros((LANES,), jnp.float32)

    def zbody(r, _):
        for c4 in range(HID // LANES):
            tmp[r, pl.ds(c4 * LANES, LANES)] = zrow
        return 0
    lax.fori_loop(0, RPT, zbody, 0)
    pltpu.sync_copy(tmp, agg.at[pl.ds(nbase, RPT)])
    plsc.subcore_barrier()

    wid = cid * 16 + sid
    ebase = wid * EPW

    def chunk_body(k, _):
        off = ebase + k * EK
        pltpu.sync_copy(src_hbm.at[pl.ds(off, EK)], s_idx)
        pltpu.sync_copy(dst_hbm.at[pl.ds(off, EK)], d_idx)
        ga = pltpu.async_copy(a_hbm.at[s_idx], a_buf, sem_a)
        gb = pltpu.async_copy(b_hbm.at[d_idx], b_buf, sem_b)
        pltpu.sync_copy(c_hbm.at[pl.ds(off, EK)], c_buf)
        ga.wait()
        gb.wait()

        def row_body(r, _):
            for c4 in range(HID // LANES):
                sl = pl.ds(c4 * LANES, LANES)
                x = a_buf[r, sl] + b_buf[r, sl] + c_buf[r, sl]
                a_buf[r, sl] = x / (1.0 + jnp.exp(-x))
            return 0
        lax.fori_loop(0, EK, row_body, 0)
        pltpu.sync_copy(a_buf, agg.at[d_idx], add=True)
        return 0
    lax.fori_loop(0, NCHUNK, chunk_body, 0)
    plsc.subcore_barrier()

    pltpu.sync_copy(agg.at[pl.ds(nbase, RPT)], tmp)
    pltpu.sync_copy(tmp, p_hbm.at[cid, pl.ds(nbase, RPT)])


@functools.cache
def _make_sc_edge():
    return pl.kernel(
        _sc_edge_body,
        out_type=jax.ShapeDtypeStruct((2, NP, HID), jnp.float32),
        mesh=plsc.VectorSubcoreMesh(core_axis_name="c", subcore_axis_name="s"),
        compiler_params=pltpu.CompilerParams(use_tc_tiling_on_sc=False),
        scratch_types=[
        pltpu.VMEM((EK,), jnp.int32),
        pltpu.VMEM((EK,), jnp.int32),
        pltpu.VMEM((EK, HID), jnp.float32),
        pltpu.VMEM((EK, HID), jnp.float32),
        pltpu.VMEM((EK, HID), jnp.float32),
        pltpu.VMEM((RPT, HID), jnp.float32),
        pltpu.VMEM_SHARED((NP, HID), jnp.float32),
        pltpu.SemaphoreType.DMA,
        pltpu.SemaphoreType.DMA,
        ],
    )


def _f32(x):
    return jax.ShapeDtypeStruct(x, jnp.float32)


def kernel(node_feat, x_t, t, edge_index, edge_attr, batch_ids, movable_mask, params):
    p = params
    tw1 = p['time_w1']
    e1 = p['enc_w1']
    m0 = p['msg_w0']
    m1 = p['msg_w1']
    u0 = p['upd_w0']
    u1 = p['upd_w1']

    t2 = t.reshape(G, 1)
    bid = batch_ids.reshape(N, 1)
    mask = movable_mask.astype(jnp.float32).reshape(N, 1)

    def row(v):
        return v.reshape(1, -1)

    h, a0, b0 = pl.pallas_call(
        _pre_body,
        out_shape=[_f32((N, HID)), _f32((N, HID)), _f32((N, HID))],
    )(t2, bid, node_feat, x_t,
      tw1[:HALF], tw1[HALF:], row(p['time_b1']), p['time_w2'], row(p['time_b2']),
      e1[:NODE_DIM], e1[NODE_DIM:NODE_DIM + 2], e1[NODE_DIM + 2:], row(p['enc_b1']),
      p['enc_w2'], row(p['enc_b2']),
      m0[:HID], m0[HID:2 * HID])

    c0, c1 = pl.pallas_call(
        _c_body,
        grid=(E // _BE,),
        in_specs=[
            pl.BlockSpec((_BE, EDGE_DIM), lambda i: (i, 0)),
            pl.BlockSpec((EDGE_DIM, HID), lambda i: (0, 0)),
            pl.BlockSpec((1, HID), lambda i: (0, 0)),
            pl.BlockSpec((EDGE_DIM, HID), lambda i: (0, 0)),
            pl.BlockSpec((1, HID), lambda i: (0, 0)),
        ],
        out_specs=[
            pl.BlockSpec((_BE, HID), lambda i: (i, 0)),
            pl.BlockSpec((_BE, HID), lambda i: (i, 0)),
        ],
        out_shape=[_f32((E, HID)), _f32((E, HID))],
    )(edge_attr, m0[2 * HID:], row(p['msg_b0']), m1[2 * HID:], row(p['msg_b1']))

    sc_edge = _make_sc_edge()
    src = edge_index[0]
    dst = edge_index[1]
    part = sc_edge(a0, b0, c0, src, dst)[:, :N, :]
    h, a1, b1 = pl.pallas_call(
        _upd_body,
        out_shape=[_f32((N, HID)), _f32((N, HID)), _f32((N, HID))],
    )(h, part[0], part[1], u0[:HID], u0[HID:], row(p['upd_b0']),
      m1[:HID], m1[HID:2 * HID])

    part = sc_edge(a1, b1, c1, src, dst)[:, :N, :]
    out = pl.pallas_call(
        _head_body,
        out_shape=_f32((N, 2)),
    )(h, part[0], part[1], u1[:HID], u1[HID:], row(p['upd_b1']),
      row(p['ln_g']), row(p['ln_b']),
      p['head_w1'], row(p['head_b1']), p['head_w2'], row(p['head_b2']),
      mask)
    return out
